# baseline (speedup 1.0000x reference)
import functools

import jax
import jax.numpy as jnp
from jax import lax
from jax.experimental import pallas as pl
from jax.experimental.pallas import tpu as pltpu
from jax.experimental.pallas import tpu_sc as plsc

_L = 16
_DEPTH = 8
_G = 4
_W = 8


def _prep_body(t, emb_ref, out_ref):
    out_ref[...] = jnp.concatenate(
        [emb_ref[1:t + 1, :], emb_ref[1:9, :]], axis=0)


def _sc_body(t, d, rows_per_w, x_hbm, ef_hbm, e0f_hbm, prep_hbm, out_hbm,
             pv_v, ef_v, e0f_v, x_v, patch_v, ring_sem):
    nc = 2
    wid = lax.axis_index("s") * nc + lax.axis_index("c")
    chunk = rows_per_w * t
    base_tok = wid * chunk
    base_row = wid * rows_per_w
    gtok = _G * t
    ngroups = rows_per_w // _G

    pltpu.sync_copy(x_hbm.at[pl.ds(base_tok, chunk)], x_v.at[pl.ds(0, chunk)])
    pltpu.sync_copy(prep_hbm, pv_v)
    pltpu.sync_copy(ef_hbm, ef_v)
    pltpu.sync_copy(e0f_hbm, e0f_v)
    eblk = pv_v.at[pl.ds(0, t)]

    def fire(b):
        pltpu.async_copy(eblk, out_hbm.at[base_row + b], ring_sem)

    def wait_one():
        pltpu.make_async_copy(
            eblk, out_hbm.at[base_row], ring_sem).wait()

    def drain_all(q):
        for i in range(_DEPTH):
            @pl.when(q > i)
            def _():
                wait_one()
        return jnp.int32(0)

    def fix_window(w, g):
        c = w // (t // _W)
        tw = (w % (t // _W)) * _W
        woff = g * gtok + c * t + tw
        xv = x_v[pl.ds(woff, _L)] == 0
        xi = jnp.where(xv, jnp.int32(1), jnp.int32(0))
        f = xi[0]
        for l in range(1, _W):
            f = f + xi[l]

        @pl.when(f > 0)
        def _():
            for j in range(_W):
                xs = x_v[pl.ds(woff + j, _L)][0]
                for k in range(d // _L):
                    ev = ef_v[pl.ds((tw + j) * d + k * _L, _L)]
                    e0v = e0f_v[pl.ds(k * _L, _L)]
                    patch_v[j, pl.ds(k * _L, _L)] = jnp.where(
                        xs == 0, e0v, ev)
            pltpu.sync_copy(
                patch_v,
                out_hbm.at[base_row + g * _G + c, pl.ds(tw, _W)])
        return g

    def group_step(g, q):
        acc = x_v[pl.ds(g * gtok, _L)] == 0
        for off in range(_L, gtok, _L):
            acc = acc | (x_v[pl.ds(g * gtok + off, _L)] == 0)
        acci = jnp.where(acc, jnp.int32(1), jnp.int32(0))
        f = acci[0]
        for l in range(1, _L):
            f = f + acci[l]

        for c in range(_G):
            q = lax.cond(q >= _DEPTH,
                         lambda qq: (wait_one(), qq - 1)[1],
                         lambda qq: qq, q)
            fire(g * _G + c)
            q = q + 1

        def slow(qq):
            qq = drain_all(qq)
            lax.fori_loop(0, _G * (t // _W), fix_window, g)
            return qq
        return lax.cond(f > 0, slow, lambda qq: qq, q)

    q = lax.fori_loop(0, ngroups, group_step, jnp.int32(0))
    drain_all(q)


def kernel(x, embeddings):
    b, t = x.shape
    v, d = embeddings.shape
    nw = 32
    rows_per_w = b // nw

    prep = pl.pallas_call(
        functools.partial(_prep_body, t),
        in_specs=[pl.BlockSpec((v, d), lambda: (0, 0))],
        out_specs=pl.BlockSpec((t + 8, d), lambda: (0, 0)),
        out_shape=jax.ShapeDtypeStruct((t + 8, d), jnp.float32),
    )(embeddings)

    mesh = plsc.VectorSubcoreMesh(core_axis_name="c", subcore_axis_name="s")
    k = functools.partial(
        pl.kernel,
        out_type=jax.ShapeDtypeStruct((b, t, d), jnp.float32),
        mesh=mesh,
        scratch_types=[
            pltpu.VMEM((t + 8, d), jnp.float32),
            pltpu.VMEM((t * d,), jnp.float32),
            pltpu.VMEM((d,), jnp.float32),
            pltpu.VMEM((rows_per_w * t + _L,), jnp.int32),
            pltpu.VMEM((_W, d), jnp.float32),
            pltpu.SemaphoreType.DMA,
        ],
    )(functools.partial(_sc_body, t, d, rows_per_w))
    return k(x.reshape(-1).astype(jnp.int32),
             embeddings[1:t + 1].reshape(-1), embeddings[0], prep)


# speedup vs baseline: 1.0236x; 1.0236x over previous
import functools

import jax
import jax.numpy as jnp
from jax import lax
from jax.experimental import pallas as pl
from jax.experimental.pallas import tpu as pltpu
from jax.experimental.pallas import tpu_sc as plsc

_L = 16
_DEPTH = 8
_G = 4


def _sc_body(t, d, rows_per_w, x_hbm, emb_hbm, out_hbm,
             pv_v, x_v, patch_v, ring_sem):
    nc = 2
    wid = lax.axis_index("s") * nc + lax.axis_index("c")
    base_row = wid * rows_per_w
    chunk = rows_per_w * t
    base_tok = wid * chunk
    ngroups = rows_per_w // _G
    nfull = t // _L
    tail = t - nfull * _L

    pltpu.sync_copy(emb_hbm.at[pl.ds(0, t + 8)], pv_v)
    pltpu.sync_copy(x_hbm.at[pl.ds(base_tok, chunk)], x_v.at[pl.ds(0, chunk)])
    eblk = pv_v.at[pl.ds(1, t)]

    offs = list(range(0, t - _L + 1, _L))
    if offs[-1] != t - _L:
        offs.append(t - _L)

    def fire(b):
        pltpu.async_copy(eblk, out_hbm.at[base_row + b], ring_sem)

    def wait_one():
        pltpu.make_async_copy(eblk, out_hbm.at[base_row], ring_sem).wait()

    def drain_all(q):
        for i in range(_DEPTH):
            @pl.when(q > i)
            def _():
                wait_one()
        return jnp.int32(0)

    def build_patch(xv, ev_row0, lane0, nvalid):
        for j in range(nvalid):
            xs = xv[lane0 + j]
            for k in range(d // _L):
                ev = pv_v[ev_row0 + j, pl.ds(k * _L, _L)]
                e0v = pv_v[0, pl.ds(k * _L, _L)]
                patch_v[j, pl.ds(k * _L, _L)] = jnp.where(xs == 0, e0v, ev)

    def fix_window(w, g):
        c = w // nfull
        wi = w % nfull
        tw = pl.multiple_of(wi * _L, _L)
        row = g * _G + c
        xv = x_v[pl.ds(row * t + tw, _L)]
        xi = jnp.where(xv == 0, jnp.int32(1), jnp.int32(0))
        f = xi[0]
        for l in range(1, _L):
            f = f + xi[l]

        @pl.when(f > 0)
        def _():
            build_patch(xv, tw + 1, 0, _L)
            pltpu.sync_copy(
                patch_v, out_hbm.at[base_row + row, pl.ds(tw, _L)])
        return g

    def fix_tail(row):
        xv = x_v[pl.ds(row * t + t - _L, _L)]
        xi = jnp.where(xv == 0, jnp.int32(1), jnp.int32(0))
        f = xi[_L - tail]
        for l in range(_L - tail + 1, _L):
            f = f + xi[l]

        @pl.when(f > 0)
        def _():
            build_patch(xv, nfull * _L + 1, _L - tail, tail)
            pltpu.sync_copy(
                patch_v.at[pl.ds(0, tail)],
                out_hbm.at[base_row + row, pl.ds(nfull * _L, tail)])

    def group_step(g, q):
        acc = x_v[pl.ds(g * _G * t + offs[0], _L)] == 0
        for c in range(_G):
            for off in offs:
                if c == 0 and off == offs[0]:
                    continue
                acc = acc | (x_v[pl.ds((g * _G + c) * t + off, _L)] == 0)
        acci = jnp.where(acc, jnp.int32(1), jnp.int32(0))
        f = acci[0]
        for l in range(1, _L):
            f = f + acci[l]

        for c in range(_G):
            q = lax.cond(q >= _DEPTH,
                         lambda qq: (wait_one(), qq - 1)[1],
                         lambda qq: qq, q)
            fire(g * _G + c)
            q = q + 1

        def slow(qq):
            qq = drain_all(qq)
            lax.fori_loop(0, _G * nfull, fix_window, g)
            for c in range(_G):
                fix_tail(g * _G + c)
            return qq
        return lax.cond(f > 0, slow, lambda qq: qq, q)

    q = lax.fori_loop(0, ngroups, group_step, jnp.int32(0))
    drain_all(q)


def kernel(x, embeddings):
    b, t = x.shape
    v, d = embeddings.shape
    nw = 32
    rows_per_w = b // nw
    mesh = plsc.VectorSubcoreMesh(core_axis_name="c", subcore_axis_name="s")
    k = functools.partial(
        pl.kernel,
        out_type=jax.ShapeDtypeStruct((b, t, d), jnp.float32),
        mesh=mesh,
        scratch_types=[
            pltpu.VMEM((t + 8, d), jnp.float32),
            pltpu.VMEM((rows_per_w * t + _L,), jnp.int32),
            pltpu.VMEM((_L, d), jnp.float32),
            pltpu.SemaphoreType.DMA,
        ],
    )(functools.partial(_sc_body, t, d, rows_per_w))
    return k(x.reshape(-1).astype(jnp.int32), embeddings)
